# baseline (speedup 1.0000x reference)
import functools
import jax
import jax.numpy as jnp
from jax import lax
from jax.experimental import pallas as pl
from jax.experimental.pallas import tpu as pltpu, tpu_sc as plsc

_N = 16384
_C = 1000
_NSTRIPE = _C // 8

_mesh = plsc.VectorSubcoreMesh(core_axis_name="c", subcore_axis_name="s")


@functools.partial(
    pl.kernel,
    mesh=_mesh,
    out_type=jax.ShapeDtypeStruct((_C, _N), jnp.float32),
    scratch_types=[
        pltpu.VMEM((8, _N // 2), jnp.float32),
        pltpu.SemaphoreType.DMA,
    ],
)
def _sc_probe(ids_hbm, out_hbm, zbuf, sem):
    wid = lax.axis_index("s") * 2 + lax.axis_index("c")
    for t in range(4):
        s = wid + 32 * t

        @pl.when(s < _NSTRIPE)
        def _():
            rows = pl.ds(s * 8, 8)
            c1 = pltpu.async_copy(zbuf, out_hbm.at[rows, : _N // 2], sem)
            c2 = pltpu.async_copy(zbuf, out_hbm.at[rows, _N // 2 :], sem)
            c1.wait()
            c2.wait()


def kernel(integers):
    out_t = _sc_probe(integers.astype(jnp.int32))
    return out_t.T


# speedup vs baseline: 1.8470x; 1.8470x over previous
import jax
import jax.numpy as jnp
from jax.experimental import pallas as pl
from jax.experimental.pallas import tpu as pltpu

_N = 16384
_C = 1000
_RC = 8
_NB = _C // _RC
_K = 8


def _onehot_block(ids_ref, out_ref, buf, sem):
    i = pl.program_id(0)
    slot = jax.lax.rem(i, _K)

    @pl.when(i >= _K)
    def _wait_prev():
        pltpu.make_async_copy(
            buf.at[slot],
            out_ref.at[pl.ds((i - _K) * _RC, _RC), :],
            sem.at[slot],
        ).wait()

    ids = ids_ref[...]
    cls = jax.lax.broadcasted_iota(jnp.int32, (_RC, _N), 0) + i * _RC
    buf[slot] = (ids == cls).astype(jnp.float32)

    pltpu.make_async_copy(
        buf.at[slot],
        out_ref.at[pl.ds(i * _RC, _RC), :],
        sem.at[slot],
    ).start()

    @pl.when(i == _NB - 1)
    def _drain():
        for j in range(_NB - _K, _NB):
            pltpu.make_async_copy(
                buf.at[j % _K],
                out_ref.at[pl.ds(j * _RC, _RC), :],
                sem.at[j % _K],
            ).wait()


def kernel(integers):
    ids = integers.astype(jnp.int32).reshape(1, _N)
    out_t = pl.pallas_call(
        _onehot_block,
        grid=(_NB,),
        in_specs=[pl.BlockSpec((1, _N), lambda i: (0, 0))],
        out_specs=pl.BlockSpec(memory_space=pl.ANY),
        out_shape=jax.ShapeDtypeStruct((_C, _N), jnp.float32),
        scratch_shapes=[
            pltpu.VMEM((_K, _RC, _N), jnp.float32),
            pltpu.SemaphoreType.DMA((_K,)),
        ],
    )(ids)
    return out_t.T
